# Initial kernel scaffold; baseline (speedup 1.0000x reference)
import jax
import jax.numpy as jnp
from jax.experimental import pallas as pl


def kernel(feat, distances, edge_index, interaction_cutoffs, rbf_kernel_means, rbf_kernel_scaling, features_to_use, bn_weight, bn_bias):
    raise NotImplementedError("write your pallas kernel here")


# speedup vs baseline: 38.5982x; 38.5982x over previous
import functools

import jax
import jax.numpy as jnp
import numpy as np
from jax import lax
from jax.experimental import pallas as pl
from jax.experimental.pallas import tpu as pltpu
from jax.experimental.pallas import tpu_sc as plsc


def _radial_and_pack_kernel(co_s, mu_s, sc_s, ftu_s, dA, dB, feat_r, he_o, p_o, *, K, T):
    for h, dref in ((0, dA), (1, dB)):
        d = dref[...]
        for k in range(K):
            g = jnp.exp(-sc_s[k] * (d - mu_s[k]) ** 2)
            c = 0.5 * (jnp.cos(np.float32(np.pi) * d / co_s[k]) + 1.0)
            g = g * jnp.where(d < co_s[k], c, jnp.float32(0.0))
            he_o[h, k] = g
    f = feat_r[...]
    tt = jnp.zeros(f.shape, jnp.int32)
    hasm = jnp.zeros(f.shape, jnp.bool_)
    for t in range(T):
        m = f == ftu_s[t]
        tt = jnp.where(m, jnp.int32(t), tt)
        hasm = hasm | m
    w = jnp.where(hasm, f, jnp.float32(0.0)).astype(jnp.int32)
    p_o[...] = w * T + tt


def _sc_scatter_kernel(he_hbm, srcE_hbm, dstE_hbm, p_hbm, o_hbm,
                       acc, srcb, dstb, peb, rowb, wbuf, vals1, outv1, idxb,
                       sem, sem2,
                       *, NT, KH, T, CH, NCHUNK, NTILE):
    cid = lax.axis_index("c")
    sid = lax.axis_index("s")
    io = lax.iota(jnp.int32, 16)
    io_q = io >> 2
    io_m = io & 3
    zero16 = jnp.zeros((16,), jnp.float32)
    CE = CH * KH
    ZC = 4000
    NZ = (NT * KH) // ZC

    def z1(i, _):
        outv1[pl.ds(i * 16, 16)] = zero16
        return _
    lax.fori_loop(0, CE // 16, z1, None)

    def zcopy(i, _):
        c = sid + i * NTILE
        pltpu.sync_copy(outv1.at[pl.ds(0, ZC)], acc.at[pl.ds(c * ZC, ZC)])
        return _
    lax.fori_loop(0, NZ // NTILE, zcopy, None)
    plsc.subcore_barrier()

    def do_chunk(c):
        eb = c * CH
        pltpu.sync_copy(srcE_hbm.at[pl.ds(eb, CH)], srcb)
        pltpu.sync_copy(dstE_hbm.at[pl.ds(eb, CH)], dstb)
        descs = [pltpu.async_copy(p_hbm.at[srcb.at[pl.ds(j * 128, 128)]],
                                  peb.at[pl.ds(j * 128, 128)], sem)
                 for j in range(CH // 128)]
        pltpu.sync_copy(he_hbm.at[cid, pl.ds(c * CE, CE)], vals1)
        for dsc in descs:
            dsc.wait()

        def idx_loop(i, _):
            pe16 = peb[pl.ds(i * 16, 16)]
            d16 = dstb[pl.ds(i * 16, 16)]
            rowb[pl.ds(i * 16, 16)] = (d16 * T + (pe16 & (T - 1))) * KH
            wbuf[pl.ds(i * 16, 16)] = (pe16 >> 3).astype(jnp.float32)
            return _
        lax.fori_loop(0, CH // 16, idx_loop, None)

        def val_loop(u, _):
            blk = 16 * (u >> 2)
            w16 = wbuf[pl.ds(blk, 16)]
            r16 = rowb[pl.ds(blk, 16)]
            perm = (u & 3) * 4 + io_q
            wl = w16.at[perm].get(mode="promise_in_bounds")
            rl = r16.at[perm].get(mode="promise_in_bounds")
            v16 = vals1[pl.ds(u * 16, 16)]
            outv1[pl.ds(u * 16, 16)] = wl * v16
            idxb[u >> 3, pl.ds((u & 7) * 16, 16)] = rl + io_m
            return _
        lax.fori_loop(0, CE // 16, val_loop, None)

        sdescs = [pltpu.async_copy(outv1.at[pl.ds(j * 128, 128)],
                                   acc.at[idxb.at[j]], sem2, add=True)
                  for j in range(CE // 128)]
        for dsc in sdescs:
            dsc.wait()

    def chunk_loop(i, _):
        c = sid + i * NTILE

        @pl.when(c < NCHUNK)
        def _():
            do_chunk(c)
        return _
    niter = (NCHUNK + NTILE - 1) // NTILE
    lax.fori_loop(0, niter, chunk_loop, None)

    plsc.subcore_barrier()

    def wcopy(i, _):
        c = sid + i * NTILE
        pltpu.sync_copy(acc.at[pl.ds(c * ZC, ZC)],
                        o_hbm.at[cid, pl.ds(c * ZC, ZC)])
        return _
    lax.fori_loop(0, NZ // NTILE, wcopy, None)


def _bn_stats_kernel(o_ref, s_o, q_o, *, T, KH):
    nb = pl.program_id(1)
    x = o_ref[0]
    r = x.shape[0] // T
    x3 = x.reshape(r, T, KH)
    s = jnp.sum(x3, axis=0)
    q = jnp.sum(x3 * x3, axis=0)

    @pl.when(nb == 0)
    def _():
        s_o[0] = s
        q_o[0] = q

    @pl.when(nb != 0)
    def _():
        s_o[0] += s
        q_o[0] += q


def _bn_apply_kernel(o_ref, sc_ref, sh_ref, out_o, *, T, KH):
    r = out_o.shape[0]
    xa = o_ref[0].reshape(r, T, KH)
    xb = o_ref[1].reshape(r, T, KH)
    x = jnp.concatenate([xa, xb], axis=2)
    out_o[...] = x * sc_ref[...] + sh_ref[...]


def kernel(feat, distances, edge_index, interaction_cutoffs, rbf_kernel_means,
           rbf_kernel_scaling, features_to_use, bn_weight, bn_bias):
    N = feat.shape[0]
    E = distances.shape[0]
    K = rbf_kernel_means.shape[0]
    T = features_to_use.shape[0]
    KH = K // 2
    NT = N * T

    f32 = jnp.float32
    i32 = jnp.int32

    d = distances.reshape(E)
    d3 = d.reshape(E // K, 2, KH)
    DH = E // 2
    dA = d3[:, 0, :].reshape(DH // 128, 128)
    dB = d3[:, 1, :].reshape(DH // 128, 128)
    NPAD = ((N + 127) // 128) * 128
    feat_r = jnp.pad(feat.reshape(N), (0, NPAD - N)).reshape(NPAD // 128, 128)

    smem_spec = pl.BlockSpec(memory_space=pltpu.SMEM)

    he_out, p_out = pl.pallas_call(
        functools.partial(_radial_and_pack_kernel, K=K, T=T),
        in_specs=[smem_spec, smem_spec, smem_spec, smem_spec,
                  pl.BlockSpec(dA.shape, lambda: (0, 0)),
                  pl.BlockSpec(dB.shape, lambda: (0, 0)),
                  pl.BlockSpec(feat_r.shape, lambda: (0, 0))],
        out_specs=[pl.BlockSpec((2, K) + dA.shape, lambda: (0, 0, 0, 0)),
                   pl.BlockSpec(feat_r.shape, lambda: (0, 0))],
        out_shape=[jax.ShapeDtypeStruct((2, K) + dA.shape, f32),
                   jax.ShapeDtypeStruct(feat_r.shape, i32)],
    )(interaction_cutoffs, rbf_kernel_means, rbf_kernel_scaling,
      features_to_use, dA, dB, feat_r)

    he_s = he_out.reshape(2, E * KH)
    p_flat = p_out.reshape(NPAD)
    srcE = edge_index[0]
    dstE = edge_index[1]

    CH = 1280
    NCHUNK = E // CH
    NTILE = 16
    CE = CH * KH
    mesh = plsc.VectorSubcoreMesh(core_axis_name="c", subcore_axis_name="s")
    sck = pl.kernel(
        functools.partial(_sc_scatter_kernel, NT=NT, KH=KH, T=T, CH=CH,
                          NCHUNK=NCHUNK, NTILE=NTILE),
        out_type=jax.ShapeDtypeStruct((2, NT * KH), f32),
        mesh=mesh,
        compiler_params=pltpu.CompilerParams(use_tc_tiling_on_sc=False),
        scratch_types=[
            pltpu.VMEM_SHARED((NT * KH,), f32),
            pltpu.VMEM((CH,), i32),
            pltpu.VMEM((CH,), i32),
            pltpu.VMEM((CH,), i32),
            pltpu.VMEM((CH,), i32),
            pltpu.VMEM((CH,), f32),
            pltpu.VMEM((CE,), f32),
            pltpu.VMEM((CE,), f32),
            pltpu.VMEM((CE // 128, 128), i32),
            pltpu.SemaphoreType.DMA,
            pltpu.SemaphoreType.DMA,
        ],
    )
    o_flat = sck(he_s, srcE, dstE, p_flat)
    o_halves = o_flat.reshape(2, NT, KH)

    NB = 50
    BR = NT // NB
    s_q = pl.pallas_call(
        functools.partial(_bn_stats_kernel, T=T, KH=KH),
        grid=(2, NB),
        in_specs=[pl.BlockSpec((1, BR, KH), lambda h, i: (h, i, 0))],
        out_specs=[pl.BlockSpec((1, T, KH), lambda h, i: (h, 0, 0)),
                   pl.BlockSpec((1, T, KH), lambda h, i: (h, 0, 0))],
        out_shape=[jax.ShapeDtypeStruct((2, T, KH), f32),
                   jax.ShapeDtypeStruct((2, T, KH), f32)],
    )(o_halves)
    S, Q = s_q

    cnt = f32(N * K)
    mean = (jnp.sum(S[0], axis=1) + jnp.sum(S[1], axis=1)) / cnt
    ex2 = (jnp.sum(Q[0], axis=1) + jnp.sum(Q[1], axis=1)) / cnt
    var = ex2 - mean * mean
    scale = bn_weight / jnp.sqrt(var + 1e-5)
    shift = bn_bias - mean * scale
    scale88 = jnp.broadcast_to(scale.reshape(1, T, 1), (1, T, K))
    shift88 = jnp.broadcast_to(shift.reshape(1, T, 1), (1, T, K))

    NB2 = 50
    R = N // NB2
    out = pl.pallas_call(
        functools.partial(_bn_apply_kernel, T=T, KH=KH),
        grid=(NB2,),
        in_specs=[pl.BlockSpec((2, R * T, KH), lambda i: (0, i, 0)),
                  pl.BlockSpec((1, T, K), lambda i: (0, 0, 0)),
                  pl.BlockSpec((1, T, K), lambda i: (0, 0, 0))],
        out_specs=pl.BlockSpec((R, T, K), lambda i: (i, 0, 0)),
        out_shape=jax.ShapeDtypeStruct((N, T, K), f32),
    )(o_halves, scale88, shift88)
    return out
